# baseline (speedup 1.0000x reference)
import functools

import jax
import jax.numpy as jnp
from jax import lax
from jax.experimental import pallas as pl
from jax.experimental.pallas import tpu as pltpu

VOCAB = 256
D = 512
XR = 4096
S = 50
RB = 16


def _tc_body(x_ref, thi_ref, tlo_ref, o_ref):
    thi = thi_ref[...]
    tlo = tlo_ref[...]
    for j in range(RB):
        row = x_ref[j]
        oh = (row[:, None] == lax.broadcasted_iota(jnp.int32, (S, VOCAB), 1)).astype(
            jnp.bfloat16
        )
        o_ref[j] = jnp.dot(oh, thi, preferred_element_type=jnp.float32) + jnp.dot(
            oh, tlo, preferred_element_type=jnp.float32
        )


@jax.jit
def _tc_lookup(x, table):
    thi = table.astype(jnp.bfloat16)
    tlo = (table - thi.astype(jnp.float32)).astype(jnp.bfloat16)
    return pl.pallas_call(
        _tc_body,
        grid=(XR // RB,),
        in_specs=[
            pl.BlockSpec((RB, S), lambda i: (i, 0)),
            pl.BlockSpec((VOCAB, D), lambda i: (0, 0)),
            pl.BlockSpec((VOCAB, D), lambda i: (0, 0)),
        ],
        out_specs=pl.BlockSpec((RB, S, D), lambda i: (i, 0, 0)),
        out_shape=jax.ShapeDtypeStruct((XR, S, D), jnp.float32),
    )(x, thi, tlo)


def kernel(x, embedding):
    return _tc_lookup(x.astype(jnp.int32), embedding)
